# baseline (speedup 1.0000x reference)
import jax
import jax.numpy as jnp
from jax import lax
from jax.experimental import pallas as pl
from jax.experimental.pallas import tpu as pltpu
from jax.experimental.pallas import tpu_sc as plsc

_E = 320000
_D = 128
_NC = 2
_NS = 16
_NW = _NC * _NS
_EPW = _E // _NW
_B = 400
_NCH = _EPW // _B
_NG = _B // 16
_GBUILD = 20
_NGATH = (_NG - _GBUILD) * 16
_NLUT = 60
_NLUTP = 64


def _sc_body(a0, a1, a2, w0, w1, w2, out, lut_hbm,
             w_v, lut_v,
             a0v0, a1v0, a2v0, a0v1, a1v1, a2v1,
             cg0, cg1, ov0, ov1,
             sem_i0, sem_i1, sem_g, sem_w0, sem_w1):
    wid = lax.axis_index("s") * _NC + lax.axis_index("c")
    base = wid * _EPW
    av = ((a0v0, a1v0, a2v0), (a0v1, a1v1, a2v1))
    cg = (cg0, cg1)
    ov = (ov0, ov1)
    sem_i = (sem_i0, sem_i1)
    sem_w = (sem_w0, sem_w1)

    pltpu.sync_copy(w0, w_v.at[pl.ds(0, 5)])
    pltpu.sync_copy(w1, w_v.at[pl.ds(5, 6)])
    pltpu.sync_copy(w2, w_v.at[pl.ds(11, 2)])

    def lut_row(r, carry):
        i = r // 12
        j = (r % 12) // 2
        k = r % 2
        for d in range(_D // 16):
            s = pl.ds(d * 16, 16)
            lut_v[r, s] = w_v[i, s] + w_v[5 + j, s] + w_v[11 + k, s]
        return carry
    lax.fori_loop(0, _NLUTP, lut_row, 0)

    pltpu.sync_copy(lut_v, lut_hbm.at[pl.ds(wid * _NLUTP, _NLUTP)])
    off = wid * _NLUTP

    def idx_start(eb, b):
        pltpu.async_copy(a0.at[pl.ds(eb, _B)], av[b][0], sem_i[b])
        pltpu.async_copy(a1.at[pl.ds(eb, _B)], av[b][1], sem_i[b])
        pltpu.async_copy(a2.at[pl.ds(eb, _B)], av[b][2], sem_i[b])

    def idx_wait(eb, b):
        pltpu.make_async_copy(a0.at[pl.ds(eb, _B)], av[b][0], sem_i[b]).wait()
        pltpu.make_async_copy(a1.at[pl.ds(eb, _B)], av[b][1], sem_i[b]).wait()
        pltpu.make_async_copy(a2.at[pl.ds(eb, _B)], av[b][2], sem_i[b]).wait()

    def gather_start(b):
        for g in range(_GBUILD, _NG):
            s = pl.ds(g * 16, 16)
            d = pl.ds((g - _GBUILD) * 16, 16)
            cg[b][d] = av[b][0][s] * 12 + av[b][1][s] * 2 + av[b][2][s] + off
        return pltpu.async_copy(
            lut_hbm.at[cg[b]],
            ov[b].at[pl.ds(_GBUILD * 16, _NGATH)], sem_g)

    def build_rows(b):
        @plsc.parallel_loop(0, _GBUILD, 1, unroll=1)
        def grp(g):
            s = pl.ds(g * 16, 16)
            c = av[b][0][s] * 12 + av[b][1][s] * 2 + av[b][2][s]
            ces = [c[l] for l in range(16)]
            for l in range(16):
                e = g * 16 + l
                for d in range(_D // 16):
                    ov[b][e, pl.ds(d * 16, 16)] = (
                        lut_v[ces[l], pl.ds(d * 16, 16)])

    def write_start(eb, b):
        pltpu.async_copy(ov[b], out.at[pl.ds(eb, _B)], sem_w[b])

    def write_wait(eb, b):
        pltpu.make_async_copy(ov[b], out.at[pl.ds(eb, _B)], sem_w[b]).wait()

    idx_start(base, 0)

    def outer(i, carry):
        for b in range(2):
            t = i * 2 + b
            eb = base + t * _B
            idx_wait(eb, b)
            idx_start(eb + _B, 1 - b)

            @pl.when(i >= 1)
            def _():
                write_wait(eb, b)

            gcp = gather_start(b)
            build_rows(b)
            gcp.wait()
            write_start(eb, b)
        return carry

    lax.fori_loop(0, (_NCH - 1) // 2, outer, 0)

    eb = base + (_NCH - 1) * _B
    idx_wait(eb, 0)
    write_wait(eb, 0)
    gcp = gather_start(0)
    build_rows(0)
    gcp.wait()
    write_start(eb, 0)

    write_wait(eb, 0)
    write_wait(eb, 1)


@jax.jit
def _run(a0, a1, a2, w0, w1, w2):
    kern = pl.kernel(
        _sc_body,
        out_type=[
            jax.ShapeDtypeStruct((_E, _D), jnp.float32),
            jax.ShapeDtypeStruct((_NW * _NLUTP, _D), jnp.float32),
        ],
        mesh=plsc.VectorSubcoreMesh(core_axis_name="c", subcore_axis_name="s"),
        compiler_params=pltpu.CompilerParams(needs_layout_passes=False),
        scratch_types=[
            pltpu.VMEM((13, _D), jnp.float32),
            pltpu.VMEM((_NLUTP, _D), jnp.float32),
            pltpu.VMEM((_B,), jnp.int32),
            pltpu.VMEM((_B,), jnp.int32),
            pltpu.VMEM((_B,), jnp.int32),
            pltpu.VMEM((_B,), jnp.int32),
            pltpu.VMEM((_B,), jnp.int32),
            pltpu.VMEM((_B,), jnp.int32),
            pltpu.VMEM((_NGATH,), jnp.int32),
            pltpu.VMEM((_NGATH,), jnp.int32),
            pltpu.VMEM((_B, _D), jnp.float32),
            pltpu.VMEM((_B, _D), jnp.float32),
            pltpu.SemaphoreType.DMA,
            pltpu.SemaphoreType.DMA,
            pltpu.SemaphoreType.DMA,
            pltpu.SemaphoreType.DMA,
            pltpu.SemaphoreType.DMA,
        ],
    )
    out, _ = kern(a0, a1, a2, w0, w1, w2)
    return out


def kernel(edge_attr, W0, W1, W2):
    a = jnp.asarray(edge_attr, jnp.int32)
    return _run(a[:, 0], a[:, 1], a[:, 2], W0, W1, W2)
